# baseline (speedup 1.0000x reference)
import jax
import jax.numpy as jnp
from jax import lax
from jax.experimental import pallas as pl
from jax.experimental.pallas import tpu as pltpu
from jax.experimental.pallas import tpu_sc as plsc

NC = 2
NS = 16
NW = NC * NS

ROWS = 16384
COLS = 26
DIM = 64
B = ROWS * COLS
CW = ROWS // NW
CH = 1024
CB = ROWS // CH
NQ = COLS * CB
QW = NQ // NW
assert QW * NW == NQ and CH % 16 == 0

MB = 8192
NMB = -(-1000000 // MB)
Z = NMB * MB


def _format_body(idxT_hbm, out_hbm, idx_v, sems):
    wid = lax.axis_index("s") * NC + lax.axis_index("c")
    col0 = wid * CW
    pltpu.sync_copy(idxT_hbm.at[:, pl.ds(col0, CW)], idx_v)
    descs = [
        pltpu.async_copy(idx_v.at[c], out_hbm.at[pl.ds(c * ROWS + col0, CW)],
                         sems.at[c])
        for c in range(COLS)
    ]
    for d in descs:
        d.wait()


@jax.jit
def _format(idxT):
    mesh = plsc.VectorSubcoreMesh(core_axis_name="c", subcore_axis_name="s")
    return pl.kernel(
        _format_body,
        out_type=jax.ShapeDtypeStruct((B,), jnp.int32),
        mesh=mesh,
        scratch_types=(
            pltpu.VMEM((COLS, CW), jnp.int32),
            pltpu.SemaphoreType.DMA((COLS,)),
        ),
        compiler_params=pltpu.CompilerParams(use_tc_tiling_on_sc=True),
    )(idxT)


def _merge_body(in_ref, out_ref):
    x = in_ref[...]
    y = x.T
    out_ref[...] = jnp.concatenate([y, y], axis=1)


@jax.jit
def _merge(tT):
    return pl.pallas_call(
        _merge_body,
        out_shape=jax.ShapeDtypeStruct((Z, 128), jnp.float32),
        grid=(NMB,),
        in_specs=[pl.BlockSpec((64, MB), lambda i: (0, i))],
        out_specs=pl.BlockSpec((MB, 128), lambda i: (i, 0)),
    )(tT)


def _gather_body(idx_hbm, table_hbm, out_hbm, idx_v, rows_v, sem):
    wid = lax.axis_index("s") * NC + lax.axis_index("c")

    def step(i, carry):
        q = wid * QW + i
        c = q // CB
        b0 = (q % CB) * CH
        pltpu.sync_copy(idx_hbm.at[pl.ds(q * CH, CH)], idx_v)

        def dbl(k, carry2):
            v = idx_v[pl.ds(k * 16, 16)]
            idx_v[pl.ds(k * 16, 16)] = v + v
            return carry2

        lax.fori_loop(0, CH // 16, dbl, 0)
        pltpu.async_copy(table_hbm.at[idx_v], rows_v, sem).wait()
        pltpu.sync_copy(rows_v, out_hbm.at[c, pl.ds(b0, CH), pl.ds(0, DIM)])
        return carry

    lax.fori_loop(0, QW, step, 0)


@jax.jit
def _gather(idx1d, table2z):
    mesh = plsc.VectorSubcoreMesh(core_axis_name="c", subcore_axis_name="s")
    return pl.kernel(
        _gather_body,
        out_type=jax.ShapeDtypeStruct((COLS, ROWS, 2 * DIM), jnp.float32),
        mesh=mesh,
        scratch_types=(
            pltpu.VMEM((CH,), jnp.int32),
            pltpu.VMEM((CH, DIM), jnp.float32),
            pltpu.SemaphoreType.DMA,
        ),
        compiler_params=pltpu.CompilerParams(use_tc_tiling_on_sc=False),
    )(idx1d, table2z)


def kernel(attributes, table):
    idxT = attributes.T.astype(jnp.int32)
    idx1d = _format(idxT)
    t2z = _merge(table.T)
    table2z = t2z.reshape(2 * Z, DIM)
    out = _gather(idx1d, table2z)
    return out[:, :, :DIM].transpose(1, 0, 2)


# speedup vs baseline: 1.0033x; 1.0033x over previous
import jax
import jax.numpy as jnp
from jax import lax
from jax.experimental import pallas as pl
from jax.experimental.pallas import tpu as pltpu
from jax.experimental.pallas import tpu_sc as plsc

NC = 2
NS = 16
NW = NC * NS

ROWS = 16384
COLS = 26
DIM = 64
B = ROWS * COLS
CW = ROWS // NW
CH = 512
CB = ROWS // CH
NQ = COLS * CB
QW = NQ // NW
assert QW * NW == NQ and CH % 16 == 0

MB = 8192
NMB = -(-1000000 // MB)
Z = NMB * MB


def _format_body(idxT_hbm, out_hbm, idx_v, sems):
    wid = lax.axis_index("s") * NC + lax.axis_index("c")
    col0 = wid * CW
    pltpu.sync_copy(idxT_hbm.at[:, pl.ds(col0, CW)], idx_v)
    descs = [
        pltpu.async_copy(idx_v.at[c], out_hbm.at[pl.ds(c * ROWS + col0, CW)],
                         sems.at[c])
        for c in range(COLS)
    ]
    for d in descs:
        d.wait()


@jax.jit
def _format(idxT):
    mesh = plsc.VectorSubcoreMesh(core_axis_name="c", subcore_axis_name="s")
    return pl.kernel(
        _format_body,
        out_type=jax.ShapeDtypeStruct((B,), jnp.int32),
        mesh=mesh,
        scratch_types=(
            pltpu.VMEM((COLS, CW), jnp.int32),
            pltpu.SemaphoreType.DMA((COLS,)),
        ),
        compiler_params=pltpu.CompilerParams(use_tc_tiling_on_sc=True),
    )(idxT)


def _merge_body(in_ref, out_ref):
    x = in_ref[...]
    y = x.T
    out_ref[...] = jnp.concatenate([y, y], axis=1)


@jax.jit
def _merge(tT):
    return pl.pallas_call(
        _merge_body,
        out_shape=jax.ShapeDtypeStruct((Z, 128), jnp.float32),
        grid=(NMB,),
        in_specs=[pl.BlockSpec((64, MB), lambda i: (0, i))],
        out_specs=pl.BlockSpec((MB, 128), lambda i: (i, 0)),
    )(tT)


def _gather_body(idx_hbm, table_hbm, out_hbm, idx_v, rows0, rows1, gsem,
                 osem0, osem1):
    wid = lax.axis_index("s") * NC + lax.axis_index("c")
    rows = (rows0, rows1)
    osem = (osem0, osem1)

    def out_slice(q):
        return out_hbm.at[q // CB, pl.ds((q % CB) * CH, CH), pl.ds(0, DIM)]

    def load_and_gather(q, buf):
        pltpu.sync_copy(idx_hbm.at[pl.ds(q * CH, CH)], idx_v)

        def dbl(k, carry2):
            v = idx_v[pl.ds(k * 16, 16)]
            idx_v[pl.ds(k * 16, 16)] = v + v
            return carry2

        lax.fori_loop(0, CH // 16, dbl, 0)
        pltpu.async_copy(table_hbm.at[idx_v], buf, gsem).wait()

    for b in range(2):
        q = wid * QW + b
        load_and_gather(q, rows[b])
        pltpu.async_copy(rows[b], out_slice(q), osem[b])

    def step2(i2, carry):
        base = 2 + i2 * 2
        for b in range(2):
            i = base + b
            q = wid * QW + i
            pltpu.make_async_copy(rows[b], out_slice(q - 2), osem[b]).wait()
            load_and_gather(q, rows[b])
            pltpu.async_copy(rows[b], out_slice(q), osem[b])
        return carry

    npairs = (QW - 2) // 2
    lax.fori_loop(0, npairs, step2, 0)
    for i in range(2 + 2 * npairs, QW):
        q = wid * QW + i
        b = i % 2
        pltpu.make_async_copy(rows[b], out_slice(q - 2), osem[b]).wait()
        load_and_gather(q, rows[b])
        pltpu.async_copy(rows[b], out_slice(q), osem[b])
    for i in range(QW - 2, QW):
        q = wid * QW + i
        b = i % 2
        pltpu.make_async_copy(rows[b], out_slice(q), osem[b]).wait()


@jax.jit
def _gather(idx1d, table2z):
    mesh = plsc.VectorSubcoreMesh(core_axis_name="c", subcore_axis_name="s")
    return pl.kernel(
        _gather_body,
        out_type=jax.ShapeDtypeStruct((COLS, ROWS, 2 * DIM), jnp.float32),
        mesh=mesh,
        scratch_types=(
            pltpu.VMEM((CH,), jnp.int32),
            pltpu.VMEM((CH, DIM), jnp.float32),
            pltpu.VMEM((CH, DIM), jnp.float32),
            pltpu.SemaphoreType.DMA,
            pltpu.SemaphoreType.DMA,
            pltpu.SemaphoreType.DMA,
        ),
        compiler_params=pltpu.CompilerParams(use_tc_tiling_on_sc=False),
    )(idx1d, table2z)


def kernel(attributes, table):
    idxT = attributes.T.astype(jnp.int32)
    idx1d = _format(idxT)
    t2z = _merge(table.T)
    table2z = t2z.reshape(2 * Z, DIM)
    out = _gather(idx1d, table2z)
    return out[:, :, :DIM].transpose(1, 0, 2)
